# baseline (speedup 1.0000x reference)
import functools

import jax
import jax.numpy as jnp
import numpy as np
from jax.experimental import pallas as pl

_R = 16
_NE = 8
_LANES = _NE * _R
_SCALING = 32.0 / 16.0
_NEG = -1e30


def _top2(l):
    iot = jax.lax.broadcasted_iota(jnp.int32, l.shape, 1).astype(jnp.float32)
    l1 = jnp.max(l, axis=1, keepdims=True)
    i1 = jnp.min(jnp.where(l == l1, iot, 8.0), axis=1, keepdims=True)
    lm = jnp.where(iot == i1, _NEG, l)
    l2 = jnp.max(lm, axis=1, keepdims=True)
    i2 = jnp.min(jnp.where(lm == l2, iot, 8.0), axis=1, keepdims=True)
    e2 = jnp.exp(l2 - l1)
    w1 = 1.0 / (1.0 + e2)
    w2 = e2 * w1
    return w1, i1, w2, i2


def _moe_kernel(x_ref, w1_ref, srep_ref, wrbt_ref, wb_ref, out_ref):
    xb = x_ref[...]
    h = jnp.dot(xb, w1_ref[...], preferred_element_type=jnp.float32)
    aa = h[:, :_LANES]
    la = h[:, _LANES:_LANES + _NE]
    w1a, i1a, w2a, i2a = _top2(la)
    lane_e = (jax.lax.broadcasted_iota(jnp.int32, aa.shape, 1) // _R
              ).astype(jnp.float32)
    ga = jnp.where(lane_e == i1a, w1a, 0.0) + jnp.where(lane_e == i2a, w2a, 0.0)
    p = aa * ga
    mid_rep = jnp.dot(p, srep_ref[...], preferred_element_type=jnp.float32,
                      precision="highest")
    lb = jnp.dot(mid_rep[:, :_R].astype(jnp.bfloat16), wrbt_ref[...],
                 preferred_element_type=jnp.float32)
    w1b, i1b, w2b, i2b = _top2(lb)
    gb = jnp.where(lane_e == i1b, w1b, 0.0) + jnp.where(lane_e == i2b, w2b, 0.0)
    z = mid_rep * gb
    out_ref[...] = jnp.dot(z, wb_ref[...], preferred_element_type=jnp.float32)


@functools.partial(jax.jit, static_argnames=())
def kernel(x, A, Bw, Wra, Wrb):
    B, S, D = x.shape
    T = B * S
    O = Bw.shape[1]
    xf = x.reshape(T, D)

    WA = A.reshape(_LANES, D).T
    W1 = jnp.concatenate(
        [WA, Wra.T, jnp.zeros((D, 256 - _LANES - _NE), x.dtype)], axis=1)

    cidx = np.arange(_LANES)
    srep = jnp.asarray(
        (cidx[:, None] % _R == cidx[None, :] % _R).astype(np.float32))
    wrbt = Wrb.T.astype(jnp.bfloat16)

    WB = Bw.transpose(0, 2, 1).reshape(_LANES, O) * _SCALING

    TB = 256
    out = pl.pallas_call(
        _moe_kernel,
        grid=(T // TB,),
        in_specs=[
            pl.BlockSpec((TB, D), lambda i: (i, 0)),
            pl.BlockSpec((D, 256), lambda i: (0, 0)),
            pl.BlockSpec((_LANES, _LANES), lambda i: (0, 0)),
            pl.BlockSpec((_R, _NE), lambda i: (0, 0)),
            pl.BlockSpec((_LANES, O), lambda i: (0, 0)),
        ],
        out_specs=pl.BlockSpec((TB, O), lambda i: (i, 0)),
        out_shape=jax.ShapeDtypeStruct((T, O), x.dtype),
    )(xf, W1, srep, wrbt, WB)
    return out.reshape(B, S, O)


# speedup vs baseline: 1.1113x; 1.1113x over previous
import functools

import jax
import jax.numpy as jnp
import numpy as np
from jax.experimental import pallas as pl

_R = 16
_NE = 8
_LANES = _NE * _R
_SCALING = 32.0 / 16.0
_NEG = -1e30


def _top2(l):
    iot = jax.lax.broadcasted_iota(jnp.int32, l.shape, 1).astype(jnp.float32)
    l1 = jnp.max(l, axis=1, keepdims=True)
    i1 = jnp.min(jnp.where(l == l1, iot, 8.0), axis=1, keepdims=True)
    lm = jnp.where(iot == i1, _NEG, l)
    l2 = jnp.max(lm, axis=1, keepdims=True)
    i2 = jnp.min(jnp.where(lm == l2, iot, 8.0), axis=1, keepdims=True)
    e2 = jnp.exp(l2 - l1)
    w1 = 1.0 / (1.0 + e2)
    w2 = e2 * w1
    return w1, i1, w2, i2


def _moe_kernel(x_ref, w1_ref, srep_ref, wrbt_ref, wb_ref, out_ref):
    xb = x_ref[...]
    h = jnp.dot(xb.astype(jnp.bfloat16), w1_ref[...],
                preferred_element_type=jnp.float32)
    aa = h[:, :_LANES]
    la = h[:, _LANES:_LANES + _NE]
    w1a, i1a, w2a, i2a = _top2(la)
    lane_e = (jax.lax.broadcasted_iota(jnp.int32, aa.shape, 1) // _R
              ).astype(jnp.float32)
    ga = jnp.where(lane_e == i1a, w1a, 0.0) + jnp.where(lane_e == i2a, w2a, 0.0)
    p = aa * ga
    mid_rep = jnp.dot(p, srep_ref[...], preferred_element_type=jnp.float32,
                      precision="highest")
    lb = jnp.dot(mid_rep[:, :_R].astype(jnp.bfloat16), wrbt_ref[...],
                 preferred_element_type=jnp.float32)
    w1b, i1b, w2b, i2b = _top2(lb)
    gb = jnp.where(lane_e == i1b, w1b, 0.0) + jnp.where(lane_e == i2b, w2b, 0.0)
    z = mid_rep * gb
    out_ref[...] = jnp.dot(z.astype(jnp.bfloat16), wb_ref[...],
                           preferred_element_type=jnp.float32)


@functools.partial(jax.jit, static_argnames=())
def kernel(x, A, Bw, Wra, Wrb):
    B, S, D = x.shape
    T = B * S
    O = Bw.shape[1]
    xf = x.reshape(T, D)

    WA = A.reshape(_LANES, D).T
    W1 = jnp.concatenate(
        [WA, Wra.T, jnp.zeros((D, 256 - _LANES - _NE), x.dtype)],
        axis=1).astype(jnp.bfloat16)

    cidx = np.arange(_LANES)
    srep = jnp.asarray(
        (cidx[:, None] % _R == cidx[None, :] % _R).astype(np.float32))
    wrbt = Wrb.T.astype(jnp.bfloat16)

    WB = (Bw.transpose(0, 2, 1).reshape(_LANES, O) * _SCALING
          ).astype(jnp.bfloat16)

    TB = 512
    out = pl.pallas_call(
        _moe_kernel,
        grid=(T // TB,),
        in_specs=[
            pl.BlockSpec((TB, D), lambda i: (i, 0)),
            pl.BlockSpec((D, 256), lambda i: (0, 0)),
            pl.BlockSpec((_LANES, _LANES), lambda i: (0, 0)),
            pl.BlockSpec((_R, _NE), lambda i: (0, 0)),
            pl.BlockSpec((_LANES, O), lambda i: (0, 0)),
        ],
        out_specs=pl.BlockSpec((TB, O), lambda i: (i, 0)),
        out_shape=jax.ShapeDtypeStruct((T, O), x.dtype),
    )(xf, W1, srep, wrbt, WB)
    return out.reshape(B, S, O)
